# Initial kernel scaffold; baseline (speedup 1.0000x reference)
import jax
import jax.numpy as jnp
from jax.experimental import pallas as pl


def kernel(scalar_representation, idx_m, W1, b1, W2, b2):
    raise NotImplementedError("write your pallas kernel here")


# speedup vs baseline: 3.0082x; 3.0082x over previous
import functools

import jax
import jax.numpy as jnp
from jax import lax
from jax.experimental import pallas as pl
from jax.experimental.pallas import tpu as pltpu
from jax.experimental.pallas import tpu_sc as plsc

N_ATOMS = 320000
N_IN = 128
N_HIDDEN = 64
N_OUT = 3
NUM_MOL = 2048


_MLP_BLOCK = 2000


def _mlp_body(x_ref, w1_ref, b1_ref, w2_ref, b2_ref, y_ref):
    x = x_ref[...]
    h = jnp.dot(x, w1_ref[...], preferred_element_type=jnp.float32)
    h = h + b1_ref[...]
    h = h * jax.nn.sigmoid(h)
    y = jnp.dot(h, w2_ref[...], preferred_element_type=jnp.float32)
    y_ref[...] = y + b2_ref[...]


def _run_mlp(x, w1, b1, w2, b2):
    grid = N_ATOMS // _MLP_BLOCK
    return pl.pallas_call(
        _mlp_body,
        grid=(grid,),
        in_specs=[
            pl.BlockSpec((_MLP_BLOCK, N_IN), lambda i: (i, 0)),
            pl.BlockSpec((N_IN, N_HIDDEN), lambda i: (0, 0)),
            pl.BlockSpec((1, N_HIDDEN), lambda i: (0, 0)),
            pl.BlockSpec((N_HIDDEN, N_OUT), lambda i: (0, 0)),
            pl.BlockSpec((1, N_OUT), lambda i: (0, 0)),
        ],
        out_specs=pl.BlockSpec((_MLP_BLOCK, N_OUT), lambda i: (i, 0)),
        out_shape=jax.ShapeDtypeStruct((N_ATOMS, N_OUT), jnp.float32),
    )(x, w1, b1.reshape(1, N_HIDDEN), w2, b2.reshape(1, N_OUT))


_NW = 32
_CHUNK = N_ATOMS // _NW
_ACC = NUM_MOL * N_OUT


def _sc_scatter_body(y_hbm, idx_hbm, out_hbm, y_v, idx_v, acc_v):
    cid = lax.axis_index("c")
    sid = lax.axis_index("s")
    wid = sid * 2 + cid
    base = wid * _CHUNK

    pltpu.sync_copy(idx_hbm.at[pl.ds(base, _CHUNK)], idx_v)
    pltpu.sync_copy(y_hbm.at[pl.ds(base * N_OUT, _CHUNK * N_OUT)], y_v)

    zeros = jnp.zeros((16,), jnp.float32)

    def zero_body(j, _):
        acc_v[pl.ds(j * 16, 16)] = zeros
        return 0

    lax.fori_loop(0, _ACC // 16, zero_body, 0, unroll=8)

    iota = lax.iota(jnp.int32, 16)
    iota3 = iota * N_OUT

    def body(i, _):
        b = i * 16
        idx16 = idx_v[pl.ds(b, 16)]
        tgt = idx16 * N_OUT
        sbase = b * N_OUT
        for c in range(N_OUT):
            src = iota3 + (sbase + c)
            yv = plsc.load_gather(y_v, [src])
            plsc.addupdate_scatter(acc_v, [tgt + c], yv)
        return 0

    lax.fori_loop(0, _CHUNK // 16, body, 0)

    pltpu.sync_copy(acc_v, out_hbm.at[wid])


def _run_sc_scatter(y_flat, idx):
    mesh = plsc.VectorSubcoreMesh(core_axis_name="c", subcore_axis_name="s")
    fn = pl.kernel(
        _sc_scatter_body,
        out_type=jax.ShapeDtypeStruct((_NW, _ACC), jnp.float32),
        mesh=mesh,
        scratch_types=[
            pltpu.VMEM((_CHUNK * N_OUT,), jnp.float32),
            pltpu.VMEM((_CHUNK,), jnp.int32),
            pltpu.VMEM((_ACC,), jnp.float32),
        ],
        compiler_params=pltpu.CompilerParams(needs_layout_passes=False),
    )
    return fn(y_flat, idx)


def _reduce_body(p_ref, o_ref):
    o_ref[...] = jnp.sum(p_ref[...], axis=0, keepdims=True)


def _run_reduce(partials):
    return pl.pallas_call(
        _reduce_body,
        out_shape=jax.ShapeDtypeStruct((1, _ACC), jnp.float32),
    )(partials)


def kernel(scalar_representation, idx_m, W1, b1, W2, b2):
    y = _run_mlp(scalar_representation, W1, b1, W2, b2)
    partials = _run_sc_scatter(y.reshape(-1), idx_m.astype(jnp.int32))
    out = _run_reduce(partials)
    return out.reshape(NUM_MOL, N_OUT)


# speedup vs baseline: 6.0276x; 2.0037x over previous
import functools

import jax
import jax.numpy as jnp
from jax import lax
from jax.experimental import pallas as pl
from jax.experimental.pallas import tpu as pltpu
from jax.experimental.pallas import tpu_sc as plsc

N_ATOMS = 320000
N_IN = 128
N_HIDDEN = 64
N_OUT = 3
NUM_MOL = 2048


_MLP_BLOCK = 3200


def _mlp_body(x_ref, w1_ref, b1_ref, w2t_ref, b2_ref, yt_ref):
    x = x_ref[...]
    h = jnp.dot(x, w1_ref[...], preferred_element_type=jnp.float32)
    h = h + b1_ref[...]
    h = h * jax.nn.sigmoid(h)
    yt = lax.dot_general(
        w2t_ref[...], h, (((1,), (1,)), ((), ())),
        preferred_element_type=jnp.float32,
    )
    yt_ref[...] = yt + b2_ref[...]


def _run_mlp(x, w1, b1, w2, b2):
    grid = N_ATOMS // _MLP_BLOCK
    return pl.pallas_call(
        _mlp_body,
        grid=(grid,),
        in_specs=[
            pl.BlockSpec((_MLP_BLOCK, N_IN), lambda i: (i, 0)),
            pl.BlockSpec((N_IN, N_HIDDEN), lambda i: (0, 0)),
            pl.BlockSpec((1, N_HIDDEN), lambda i: (0, 0)),
            pl.BlockSpec((N_OUT, N_HIDDEN), lambda i: (0, 0)),
            pl.BlockSpec((N_OUT, 1), lambda i: (0, 0)),
        ],
        out_specs=pl.BlockSpec((N_OUT, _MLP_BLOCK), lambda i: (0, i)),
        out_shape=jax.ShapeDtypeStruct((N_OUT, N_ATOMS), jnp.float32),
    )(x, w1, b1.reshape(1, N_HIDDEN), w2.T, b2.reshape(N_OUT, 1))


_NW = 32
_CHUNK = N_ATOMS // _NW
_ACC = NUM_MOL * N_OUT


def _sc_scatter_body(y_hbm, idx_hbm, out_hbm, y_v, idx_v, acc_v):
    cid = lax.axis_index("c")
    sid = lax.axis_index("s")
    wid = sid * 2 + cid
    base = wid * _CHUNK

    pltpu.sync_copy(idx_hbm.at[pl.ds(base, _CHUNK)], idx_v)
    for c in range(N_OUT):
        pltpu.sync_copy(
            y_hbm.at[pl.ds(c * N_ATOMS + base, _CHUNK)],
            y_v.at[pl.ds(c * _CHUNK, _CHUNK)],
        )

    zeros = jnp.zeros((16,), jnp.float32)

    def zero_body(j, _):
        acc_v[pl.ds(j * 16, 16)] = zeros
        return 0

    lax.fori_loop(0, _ACC // 16, zero_body, 0, unroll=8)

    def body(i, _):
        b = i * 16
        idx16 = idx_v[pl.ds(b, 16)]
        tgt = idx16 * N_OUT
        for c in range(N_OUT):
            yv = y_v[pl.ds(c * _CHUNK + b, 16)]
            plsc.addupdate_scatter(acc_v, [tgt + c], yv)
        return 0

    lax.fori_loop(0, _CHUNK // 16, body, 0, unroll=4)

    pltpu.sync_copy(acc_v, out_hbm.at[wid])


def _run_sc_scatter(y_flat, idx):
    mesh = plsc.VectorSubcoreMesh(core_axis_name="c", subcore_axis_name="s")
    fn = pl.kernel(
        _sc_scatter_body,
        out_type=jax.ShapeDtypeStruct((_NW, _ACC), jnp.float32),
        mesh=mesh,
        scratch_types=[
            pltpu.VMEM((_CHUNK * N_OUT,), jnp.float32),
            pltpu.VMEM((_CHUNK,), jnp.int32),
            pltpu.VMEM((_ACC,), jnp.float32),
        ],
        compiler_params=pltpu.CompilerParams(needs_layout_passes=False),
    )
    return fn(y_flat, idx)


def _reduce_body(p_ref, o_ref):
    o_ref[...] = jnp.sum(p_ref[...], axis=0, keepdims=True)


def _run_reduce(partials):
    return pl.pallas_call(
        _reduce_body,
        out_shape=jax.ShapeDtypeStruct((1, _ACC), jnp.float32),
    )(partials)


def kernel(scalar_representation, idx_m, W1, b1, W2, b2):
    y = _run_mlp(scalar_representation, W1, b1, W2, b2)
    partials = _run_sc_scatter(y.reshape(-1), idx_m.astype(jnp.int32))
    out = _run_reduce(partials)
    return out.reshape(NUM_MOL, N_OUT)


# speedup vs baseline: 7.0863x; 1.1756x over previous
import functools

import jax
import jax.numpy as jnp
from jax import lax
from jax.experimental import pallas as pl
from jax.experimental.pallas import tpu as pltpu
from jax.experimental.pallas import tpu_sc as plsc

N_ATOMS = 320000
N_IN = 128
N_HIDDEN = 64
N_OUT = 3
NUM_MOL = 2048


_MLP_BLOCK = 6400


def _mlp_body(x_ref, w1_ref, b1_ref, w2t_ref, b2_ref, yt_ref):
    x = x_ref[...]
    h = jnp.dot(x, w1_ref[...], preferred_element_type=jnp.float32)
    h = h + b1_ref[...]
    h = h * jax.nn.sigmoid(h)
    yt = lax.dot_general(
        w2t_ref[...], h, (((1,), (1,)), ((), ())),
        preferred_element_type=jnp.float32,
    )
    yt_ref[...] = yt + b2_ref[...]


def _run_mlp(x, w1, b1, w2, b2):
    grid = N_ATOMS // _MLP_BLOCK
    return pl.pallas_call(
        _mlp_body,
        grid=(grid,),
        in_specs=[
            pl.BlockSpec((_MLP_BLOCK, N_IN), lambda i: (i, 0)),
            pl.BlockSpec((N_IN, N_HIDDEN), lambda i: (0, 0)),
            pl.BlockSpec((1, N_HIDDEN), lambda i: (0, 0)),
            pl.BlockSpec((N_OUT, N_HIDDEN), lambda i: (0, 0)),
            pl.BlockSpec((N_OUT, 1), lambda i: (0, 0)),
        ],
        out_specs=pl.BlockSpec((N_OUT, _MLP_BLOCK), lambda i: (0, i)),
        out_shape=jax.ShapeDtypeStruct((N_OUT, N_ATOMS), jnp.float32),
    )(x, w1, b1.reshape(1, N_HIDDEN), w2.T, b2.reshape(N_OUT, 1))


_NW = 32
_CHUNK = N_ATOMS // _NW
_ACC = NUM_MOL * N_OUT
_BANKS = 5


def _sc_scatter_body(y_hbm, idx_hbm, out_hbm, y_v, idx_v, acc_v):
    cid = lax.axis_index("c")
    sid = lax.axis_index("s")
    wid = sid * 2 + cid
    base = wid * _CHUNK

    pltpu.sync_copy(idx_hbm.at[pl.ds(base, _CHUNK)], idx_v)
    for c in range(N_OUT):
        pltpu.sync_copy(
            y_hbm.at[pl.ds(c * N_ATOMS + base, _CHUNK)],
            y_v.at[pl.ds(c * _CHUNK, _CHUNK)],
        )

    zeros = jnp.zeros((16,), jnp.float32)

    def zero_body(j, _):
        acc_v[pl.ds(j * 16, 16)] = zeros
        return 0

    lax.fori_loop(0, _BANKS * _ACC // 16, zero_body, 0, unroll=8)

    def body(i, _):
        for j in range(_BANKS):
            b = (i * _BANKS + j) * 16
            idx16 = idx_v[pl.ds(b, 16)]
            tgt = idx16 * N_OUT + (j * _ACC)
            for c in range(N_OUT):
                yv = y_v[pl.ds(c * _CHUNK + b, 16)]
                plsc.addupdate_scatter(acc_v, [tgt + c], yv)
        return 0

    lax.fori_loop(0, _CHUNK // (16 * _BANKS), body, 0)

    def merge_body(j, _):
        s = acc_v[pl.ds(j * 16, 16)]
        for k in range(1, _BANKS):
            s = s + acc_v[pl.ds(k * _ACC + j * 16, 16)]
        acc_v[pl.ds(j * 16, 16)] = s
        return 0

    lax.fori_loop(0, _ACC // 16, merge_body, 0, unroll=4)

    pltpu.sync_copy(acc_v.at[pl.ds(0, _ACC)], out_hbm.at[wid])


def _run_sc_scatter(y_flat, idx):
    mesh = plsc.VectorSubcoreMesh(core_axis_name="c", subcore_axis_name="s")
    fn = pl.kernel(
        _sc_scatter_body,
        out_type=jax.ShapeDtypeStruct((_NW, _ACC), jnp.float32),
        mesh=mesh,
        scratch_types=[
            pltpu.VMEM((_CHUNK * N_OUT,), jnp.float32),
            pltpu.VMEM((_CHUNK,), jnp.int32),
            pltpu.VMEM((_BANKS * _ACC,), jnp.float32),
        ],
        compiler_params=pltpu.CompilerParams(needs_layout_passes=False),
    )
    return fn(y_flat, idx)


def _reduce_body(p_ref, o_ref):
    o_ref[...] = jnp.sum(p_ref[...], axis=0, keepdims=True)


def _run_reduce(partials):
    return pl.pallas_call(
        _reduce_body,
        out_shape=jax.ShapeDtypeStruct((1, _ACC), jnp.float32),
    )(partials)


def kernel(scalar_representation, idx_m, W1, b1, W2, b2):
    y = _run_mlp(scalar_representation, W1, b1, W2, b2)
    partials = _run_sc_scatter(y.reshape(-1), idx_m.astype(jnp.int32))
    out = _run_reduce(partials)
    return out.reshape(NUM_MOL, N_OUT)


# speedup vs baseline: 7.8782x; 1.1118x over previous
import functools

import jax
import jax.numpy as jnp
from jax import lax
from jax.experimental import pallas as pl
from jax.experimental.pallas import tpu as pltpu
from jax.experimental.pallas import tpu_sc as plsc

N_ATOMS = 320000
N_IN = 128
N_HIDDEN = 64
N_OUT = 3
NUM_MOL = 2048


_MLP_BLOCK = 12800


def _mlp_body(x_ref, w1_ref, b1_ref, w2t_ref, b2_ref, yt_ref):
    x = x_ref[...]
    h = jnp.dot(x, w1_ref[...], preferred_element_type=jnp.float32)
    h = h + b1_ref[...]
    h = h * (0.5 * jnp.tanh(0.5 * h) + 0.5)
    yt = lax.dot_general(
        w2t_ref[...], h, (((1,), (1,)), ((), ())),
        preferred_element_type=jnp.float32,
    )
    yt_ref[...] = yt + b2_ref[...]


def _run_mlp(x, w1, b1, w2, b2):
    grid = N_ATOMS // _MLP_BLOCK
    return pl.pallas_call(
        _mlp_body,
        grid=(grid,),
        in_specs=[
            pl.BlockSpec((_MLP_BLOCK, N_IN), lambda i: (i, 0)),
            pl.BlockSpec((N_IN, N_HIDDEN), lambda i: (0, 0)),
            pl.BlockSpec((1, N_HIDDEN), lambda i: (0, 0)),
            pl.BlockSpec((N_OUT, N_HIDDEN), lambda i: (0, 0)),
            pl.BlockSpec((N_OUT, 1), lambda i: (0, 0)),
        ],
        out_specs=pl.BlockSpec((N_OUT, _MLP_BLOCK), lambda i: (0, i)),
        out_shape=jax.ShapeDtypeStruct((N_OUT, N_ATOMS), jnp.float32),
    )(x, w1, b1.reshape(1, N_HIDDEN), w2.T, b2.reshape(N_OUT, 1))


_NW = 32
_CHUNK = N_ATOMS // _NW
_ACC = NUM_MOL * N_OUT
_BANKS = 5


def _sc_scatter_body(y_hbm, idx_hbm, out_hbm, y_v, idx_v, acc_v):
    cid = lax.axis_index("c")
    sid = lax.axis_index("s")
    wid = sid * 2 + cid
    base = wid * _CHUNK

    pltpu.sync_copy(idx_hbm.at[pl.ds(base, _CHUNK)], idx_v)
    for c in range(N_OUT):
        pltpu.sync_copy(
            y_hbm.at[pl.ds(c * N_ATOMS + base, _CHUNK)],
            y_v.at[pl.ds(c * _CHUNK, _CHUNK)],
        )

    zeros = jnp.zeros((16,), jnp.float32)

    def zero_body(j, _):
        acc_v[pl.ds(j * 16, 16)] = zeros
        return 0

    lax.fori_loop(0, _BANKS * _ACC // 16, zero_body, 0, unroll=8)

    def body(i, _):
        for j in range(_BANKS):
            b = (i * _BANKS + j) * 16
            idx16 = idx_v[pl.ds(b, 16)]
            tgt = idx16 * N_OUT + (j * _ACC)
            for c in range(N_OUT):
                yv = y_v[pl.ds(c * _CHUNK + b, 16)]
                plsc.addupdate_scatter(acc_v, [tgt + c], yv)
        return 0

    lax.fori_loop(0, _CHUNK // (16 * _BANKS), body, 0)

    def merge_body(j, _):
        s = acc_v[pl.ds(j * 16, 16)]
        for k in range(1, _BANKS):
            s = s + acc_v[pl.ds(k * _ACC + j * 16, 16)]
        acc_v[pl.ds(j * 16, 16)] = s
        return 0

    lax.fori_loop(0, _ACC // 16, merge_body, 0, unroll=4)

    pltpu.sync_copy(acc_v.at[pl.ds(0, _ACC)], out_hbm.at[wid])


def _run_sc_scatter(y_flat, idx):
    mesh = plsc.VectorSubcoreMesh(core_axis_name="c", subcore_axis_name="s")
    fn = pl.kernel(
        _sc_scatter_body,
        out_type=jax.ShapeDtypeStruct((_NW, _ACC), jnp.float32),
        mesh=mesh,
        scratch_types=[
            pltpu.VMEM((_CHUNK * N_OUT,), jnp.float32),
            pltpu.VMEM((_CHUNK,), jnp.int32),
            pltpu.VMEM((_BANKS * _ACC,), jnp.float32),
        ],
        compiler_params=pltpu.CompilerParams(needs_layout_passes=False),
    )
    return fn(y_flat, idx)


def _reduce_body(p_ref, o_ref):
    o_ref[...] = jnp.sum(p_ref[...], axis=0, keepdims=True)


def _run_reduce(partials):
    return pl.pallas_call(
        _reduce_body,
        out_shape=jax.ShapeDtypeStruct((1, _ACC), jnp.float32),
    )(partials)


def kernel(scalar_representation, idx_m, W1, b1, W2, b2):
    y = _run_mlp(scalar_representation, W1, b1, W2, b2)
    partials = _run_sc_scatter(y.reshape(-1), idx_m.astype(jnp.int32))
    out = _run_reduce(partials)
    return out.reshape(NUM_MOL, N_OUT)


# speedup vs baseline: 9.0638x; 1.1505x over previous
import functools

import jax
import jax.numpy as jnp
from jax import lax
from jax.experimental import pallas as pl
from jax.experimental.pallas import tpu as pltpu
from jax.experimental.pallas import tpu_sc as plsc

N_ATOMS = 320000
N_IN = 128
N_HIDDEN = 64
N_OUT = 3
NUM_MOL = 2048


_MLP_BLOCK = 12800


def _mlp_body(x_ref, w1_ref, b1_ref, w2t_ref, b2_ref, yt_ref):
    x = x_ref[...]
    h = jnp.dot(x, w1_ref[...], preferred_element_type=jnp.float32)
    h = h + b1_ref[...]
    h = h * (0.5 * jnp.tanh(0.5 * h) + 0.5)
    yt = lax.dot_general(
        w2t_ref[...], h, (((1,), (1,)), ((), ())),
        preferred_element_type=jnp.float32,
    )
    yt_ref[...] = yt + b2_ref[...]


def _run_mlp(x, w1, b1, w2, b2):
    grid = N_ATOMS // _MLP_BLOCK
    return pl.pallas_call(
        _mlp_body,
        grid=(grid,),
        in_specs=[
            pl.BlockSpec((_MLP_BLOCK, N_IN), lambda i: (i, 0)),
            pl.BlockSpec((N_IN, N_HIDDEN), lambda i: (0, 0)),
            pl.BlockSpec((1, N_HIDDEN), lambda i: (0, 0)),
            pl.BlockSpec((N_OUT, N_HIDDEN), lambda i: (0, 0)),
            pl.BlockSpec((N_OUT, 1), lambda i: (0, 0)),
        ],
        out_specs=pl.BlockSpec((N_OUT, _MLP_BLOCK), lambda i: (0, i)),
        out_shape=jax.ShapeDtypeStruct((N_OUT, N_ATOMS), jnp.float32),
    )(x, w1, b1.reshape(1, N_HIDDEN), w2.T, b2.reshape(N_OUT, 1))


_NW = 32
_CHUNK = N_ATOMS // _NW
_ACC = NUM_MOL * N_OUT


def _sc_scatter_body(y_hbm, idx_hbm, out_hbm, y_v, idx_v, acc_v):
    cid = lax.axis_index("c")
    sid = lax.axis_index("s")
    wid = sid * 2 + cid
    base = wid * _CHUNK

    pltpu.sync_copy(idx_hbm.at[pl.ds(base, _CHUNK)], idx_v.at[pl.ds(0, _CHUNK)])
    for c in range(N_OUT):
        pltpu.sync_copy(
            y_hbm.at[pl.ds(c * N_ATOMS + base, _CHUNK)],
            y_v.at[pl.ds(c * _CHUNK, _CHUNK)],
        )

    zeros = jnp.zeros((16,), jnp.float32)

    def zero_body(j, _):
        acc_v[pl.ds(j * 16, 16)] = zeros
        return 0

    lax.fori_loop(0, _ACC // 16, zero_body, 0, unroll=8)

    iota = lax.iota(jnp.int32, 16)
    last_lane = iota == 15
    not_last = iota != 15

    def body(i, _):
        b = i * 16
        idx16 = idx_v[pl.ds(b, 16)]
        idxp1 = idx_v[pl.ds(b + 1, 16)]
        neq = idx16 != idxp1
        m_add = neq | last_lane
        m_sub = neq & not_last
        tgt = idx16 * N_OUT
        tgtp1 = idxp1 * N_OUT
        for c in range(N_OUT):
            yv = y_v[pl.ds(c * _CHUNK + b, 16)]
            s = plsc.cumsum(yv)
            plsc.addupdate_scatter(acc_v, [tgt + c], s, mask=m_add)
            plsc.addupdate_scatter(acc_v, [tgtp1 + c], -s, mask=m_sub)
        return 0

    lax.fori_loop(0, _CHUNK // 16, body, 0, unroll=2)

    pltpu.sync_copy(acc_v, out_hbm.at[wid])


def _run_sc_scatter(y_flat, idx):
    mesh = plsc.VectorSubcoreMesh(core_axis_name="c", subcore_axis_name="s")
    fn = pl.kernel(
        _sc_scatter_body,
        out_type=jax.ShapeDtypeStruct((_NW, _ACC), jnp.float32),
        mesh=mesh,
        scratch_types=[
            pltpu.VMEM((_CHUNK * N_OUT,), jnp.float32),
            pltpu.VMEM((_CHUNK + 16,), jnp.int32),
            pltpu.VMEM((_ACC,), jnp.float32),
        ],
        compiler_params=pltpu.CompilerParams(needs_layout_passes=False),
    )
    return fn(y_flat, idx)


def _reduce_body(p_ref, o_ref):
    o_ref[...] = jnp.sum(p_ref[...], axis=0, keepdims=True)


def _run_reduce(partials):
    return pl.pallas_call(
        _reduce_body,
        out_shape=jax.ShapeDtypeStruct((1, _ACC), jnp.float32),
    )(partials)


def kernel(scalar_representation, idx_m, W1, b1, W2, b2):
    y = _run_mlp(scalar_representation, W1, b1, W2, b2)
    partials = _run_sc_scatter(y.reshape(-1), idx_m.astype(jnp.int32))
    out = _run_reduce(partials)
    return out.reshape(NUM_MOL, N_OUT)


# speedup vs baseline: 9.2669x; 1.0224x over previous
import functools

import jax
import jax.numpy as jnp
from jax import lax
from jax.experimental import pallas as pl
from jax.experimental.pallas import tpu as pltpu
from jax.experimental.pallas import tpu_sc as plsc

N_ATOMS = 320000
N_IN = 128
N_HIDDEN = 64
N_OUT = 3
NUM_MOL = 2048


_MLP_BLOCK = 16000


def _mlp_body(x_ref, w1_ref, b1_ref, w2t_ref, b2_ref, yt_ref):
    x = x_ref[...]
    h = jnp.dot(x, w1_ref[...], preferred_element_type=jnp.float32)
    h = h + b1_ref[...]
    h = h * (0.5 * jnp.tanh(0.5 * h) + 0.5)
    yt = lax.dot_general(
        w2t_ref[...], h, (((1,), (1,)), ((), ())),
        preferred_element_type=jnp.float32,
    )
    yt_ref[...] = yt + b2_ref[...]


def _run_mlp(x, w1, b1, w2, b2):
    grid = N_ATOMS // _MLP_BLOCK
    return pl.pallas_call(
        _mlp_body,
        grid=(grid,),
        in_specs=[
            pl.BlockSpec((_MLP_BLOCK, N_IN), lambda i: (i, 0)),
            pl.BlockSpec((N_IN, N_HIDDEN), lambda i: (0, 0)),
            pl.BlockSpec((1, N_HIDDEN), lambda i: (0, 0)),
            pl.BlockSpec((N_OUT, N_HIDDEN), lambda i: (0, 0)),
            pl.BlockSpec((N_OUT, 1), lambda i: (0, 0)),
        ],
        out_specs=pl.BlockSpec((N_OUT, _MLP_BLOCK), lambda i: (0, i)),
        out_shape=jax.ShapeDtypeStruct((N_OUT, N_ATOMS), jnp.float32),
    )(x, w1, b1.reshape(1, N_HIDDEN), w2.T, b2.reshape(N_OUT, 1))


_NW = 32
_CHUNK = N_ATOMS // _NW
_ACC = NUM_MOL * N_OUT


def _sc_scatter_body(y_hbm, idx_hbm, out_hbm, y_v, idx_v, acc_v):
    cid = lax.axis_index("c")
    sid = lax.axis_index("s")
    wid = sid * 2 + cid
    base = wid * _CHUNK

    pltpu.sync_copy(idx_hbm.at[pl.ds(base, _CHUNK)], idx_v.at[pl.ds(0, _CHUNK)])
    for c in range(N_OUT):
        pltpu.sync_copy(
            y_hbm.at[pl.ds(c * N_ATOMS + base, _CHUNK)],
            y_v.at[pl.ds(c * _CHUNK, _CHUNK)],
        )

    zeros = jnp.zeros((16,), jnp.float32)

    def zero_body(j, _):
        acc_v[pl.ds(j * 16, 16)] = zeros
        return 0

    lax.fori_loop(0, _ACC // 16, zero_body, 0, unroll=8)

    iota = lax.iota(jnp.int32, 16)
    last_lane = iota == 15
    not_last = iota != 15

    def body(i, _):
        b = i * 16
        idx16 = idx_v[pl.ds(b, 16)]
        idxp1 = idx_v[pl.ds(b + 1, 16)]
        neq = idx16 != idxp1
        m_add = neq | last_lane
        m_sub = neq & not_last
        tgt = idx16 * N_OUT
        tgtp1 = idxp1 * N_OUT
        for c in range(N_OUT):
            yv = y_v[pl.ds(c * _CHUNK + b, 16)]
            s = plsc.cumsum(yv)
            plsc.addupdate_scatter(acc_v, [tgt + c], s, mask=m_add)
            plsc.addupdate_scatter(acc_v, [tgtp1 + c], -s, mask=m_sub)
        return 0

    lax.fori_loop(0, _CHUNK // 16, body, 0, unroll=2)

    pltpu.sync_copy(acc_v, out_hbm.at[wid])


def _run_sc_scatter(y_flat, idx):
    mesh = plsc.VectorSubcoreMesh(core_axis_name="c", subcore_axis_name="s")
    fn = pl.kernel(
        _sc_scatter_body,
        out_type=jax.ShapeDtypeStruct((_NW, _ACC), jnp.float32),
        mesh=mesh,
        scratch_types=[
            pltpu.VMEM((_CHUNK * N_OUT,), jnp.float32),
            pltpu.VMEM((_CHUNK + 16,), jnp.int32),
            pltpu.VMEM((_ACC,), jnp.float32),
        ],
        compiler_params=pltpu.CompilerParams(needs_layout_passes=False),
    )
    return fn(y_flat, idx)


def _reduce_body(p_ref, o_ref):
    o_ref[...] = jnp.sum(p_ref[...], axis=0, keepdims=True)


def _run_reduce(partials):
    return pl.pallas_call(
        _reduce_body,
        out_shape=jax.ShapeDtypeStruct((1, _ACC), jnp.float32),
    )(partials)


def kernel(scalar_representation, idx_m, W1, b1, W2, b2):
    y = _run_mlp(scalar_representation, W1, b1, W2, b2)
    partials = _run_sc_scatter(y.reshape(-1), idx_m.astype(jnp.int32))
    out = _run_reduce(partials)
    return out.reshape(NUM_MOL, N_OUT)
